# baseline (speedup 1.0000x reference)
import jax
import jax.numpy as jnp
from jax import lax
from jax.experimental import pallas as pl
from jax.experimental.pallas import tpu as pltpu
from jax.experimental.pallas import tpu_sc as plsc

B, S, D = 1, 2048, 2048
E, TOP_K = 8, 2
DFF = 1408
DSH = 5632
T = B * S
A = T * TOP_K

M = 256
NT = A // M + E
NTOT = NT * M

BTS = 256
BF = 512
NF = DSH // BF

NW = 32


def _router_body(x_ref, gw_ref, sgw_ref, logits_ref, w_ref, gate_ref,
                 dest_ref, te_ref, na_ref):
    xf = x_ref[...]
    x = xf.astype(jnp.bfloat16)
    logits = lax.dot_general(
        x, gw_ref[...].astype(jnp.bfloat16), (((1,), (0,)), ((), ())),
        preferred_element_type=jnp.float32)
    logits_ref[...] = logits
    gate_ref[...] = jax.nn.sigmoid(
        jnp.sum(xf * sgw_ref[...], axis=-1, keepdims=True))
    mx = jnp.max(logits, axis=-1, keepdims=True)
    ex = jnp.exp(logits - mx)
    p = ex / jnp.sum(ex, axis=-1, keepdims=True)
    lane = lax.broadcasted_iota(jnp.int32, p.shape, 1)
    m1 = jnp.max(p, axis=-1, keepdims=True)
    i1 = jnp.min(jnp.where(p >= m1, lane, E), axis=-1, keepdims=True)
    oh1 = lane == i1
    p2 = jnp.where(oh1, -1.0, p)
    m2 = jnp.max(p2, axis=-1, keepdims=True)
    i2 = jnp.min(jnp.where(p2 >= m2, lane, E), axis=-1, keepdims=True)
    oh2 = lane == i2
    denom = m1 + m2
    w_ref[...] = jnp.concatenate([m1 / denom, m2 / denom], axis=1)

    oh = jnp.concatenate([oh1, oh2], axis=0).astype(jnp.float32)
    inc = oh
    sh = 1
    while sh < A:
        inc = inc + jnp.concatenate(
            [jnp.zeros((sh, E), jnp.float32), inc[:-sh]], axis=0)
        sh *= 2
    r = inc - oh
    counts = inc[A - 1:A, :]
    tpe = jnp.floor((counts + (M - 1)) / M)
    ts_inc = tpe
    sh = 1
    while sh < E:
        ts_inc = ts_inc + jnp.concatenate(
            [jnp.zeros((1, sh), jnp.float32), ts_inc[:, :-sh]], axis=1)
        sh *= 2
    ts = ts_inc - tpe
    dest = jnp.sum(oh * (ts * M + r), axis=1, keepdims=True)
    dest_ref[...] = dest.astype(jnp.int32)

    tile = lax.broadcasted_iota(jnp.int32, (NT, E), 0).astype(jnp.float32)
    cmp = (tile >= jnp.broadcast_to(ts, (NT, E))).astype(jnp.float32)
    te = jnp.sum(cmp, axis=1, keepdims=True) - 1.0
    te_ref[...] = jnp.maximum(te, 0.0).astype(jnp.int32)
    na_ref[...] = ts_inc[:, E - 1:E].astype(jnp.int32)


def _conv_body(in_ref, out_ref):
    out_ref[...] = in_ref[...].astype(jnp.bfloat16)


def _to_bf16(a, blk):
    n = a.shape[0] // blk
    return pl.pallas_call(
        _conv_body,
        grid=(n,),
        in_specs=[pl.BlockSpec((blk,) + a.shape[1:],
                               lambda i: (i,) + (0,) * (a.ndim - 1))],
        out_specs=pl.BlockSpec((blk,) + a.shape[1:],
                               lambda i: (i,) + (0,) * (a.ndim - 1)),
        out_shape=jax.ShapeDtypeStruct(a.shape, jnp.bfloat16),
        compiler_params=pltpu.CompilerParams(
            dimension_semantics=("arbitrary",)),
    )(a)


def _gu_body(te_ref, na_ref, xs_ref, wg_ref, wu_ref, h_ref):
    j = pl.program_id(0)

    @pl.when(j < na_ref[0, 0])
    def _():
        xb = xs_ref[...]
        g = jnp.dot(xb, wg_ref[0], preferred_element_type=jnp.float32)
        u = jnp.dot(xb, wu_ref[0], preferred_element_type=jnp.float32)
        h_ref[...] = g * jax.nn.sigmoid(g) * u


def _dn_body(te_ref, na_ref, h_ref, wd_ref, ys_ref):
    j = pl.program_id(0)

    @pl.when(j < na_ref[0, 0])
    def _():
        h = h_ref[...]
        ys_ref[...] = jax.lax.dot_general(
            h, wd_ref[0], (((1,), (0,)), ((), ())),
            preferred_element_type=jnp.float32)


def _shared_body(x_ref, wg_ref, wu_ref, wd_ref, gate_ref, g0_ref, g1_ref,
                 w_ref, out_ref):
    f = pl.program_id(1)
    x = x_ref[...]
    g = jnp.dot(x, wg_ref[...], preferred_element_type=jnp.float32)
    u = jnp.dot(x, wu_ref[...], preferred_element_type=jnp.float32)
    h = g * jax.nn.sigmoid(g) * u
    partial = jnp.dot(h, wd_ref[...], preferred_element_type=jnp.float32)
    prev = jnp.where(f == 0, jnp.zeros_like(partial), out_ref[...])
    acc = prev + partial

    @pl.when(f < NF - 1)
    def _():
        out_ref[...] = acc

    @pl.when(f == NF - 1)
    def _():
        wv = w_ref[...]
        out_ref[...] = (gate_ref[...] * acc + wv[:, 0:1] * g0_ref[...]
                        + wv[:, 1:2] * g1_ref[...])


def _sc_scatter(x_ref, dest_ref, xs_ref, idx_v, rows_v, sem):
    w = lax.axis_index("s") * 2 + lax.axis_index("c")
    t0 = (w % 16) * 128
    pltpu.sync_copy(dest_ref.at[w], idx_v)
    for c in range(4):
        pltpu.sync_copy(x_ref.at[pl.ds(t0 + c * 32, 32)], rows_v)
        pltpu.async_copy(rows_v, xs_ref.at[idx_v.at[c]], sem).wait()


def _sc_gather(ys_ref, pos_ref, g0_ref, g1_ref, idx_v, rows_v, sem):
    w = lax.axis_index("s") * 2 + lax.axis_index("c")
    for k in range(2):
        gout = g0_ref if k == 0 else g1_ref
        for c in range(2):
            j = w * 2 + c
            pltpu.sync_copy(pos_ref.at[k, j], idx_v)
            pltpu.async_copy(ys_ref.at[idx_v], rows_v, sem).wait()
            pltpu.sync_copy(rows_v, gout.at[pl.ds(j * 32, 32)])


@jax.jit
def kernel(hidden_states, gate_w, Wg, Wu, Wd, sWg, sWu, sWd, shared_gate_w):
    x = hidden_states.reshape(T, D)
    sgw_t = shared_gate_w.reshape(1, D)
    logits, w, gate, dest, te, na = pl.pallas_call(
        _router_body,
        out_shape=(
            jax.ShapeDtypeStruct((T, E), jnp.float32),
            jax.ShapeDtypeStruct((T, TOP_K), jnp.float32),
            jax.ShapeDtypeStruct((T, 1), jnp.float32),
            jax.ShapeDtypeStruct((A, 1), jnp.int32),
            jax.ShapeDtypeStruct((NT, 1), jnp.int32),
            jax.ShapeDtypeStruct((1, 1), jnp.int32),
        ),
    )(x, gate_w, sgw_t)

    dest_flat = dest.reshape(A)
    dest_sc = dest_flat.reshape(NW, 4, 32)
    pos_sc = dest_flat.reshape(2, 64, 32)

    sc_mesh = plsc.VectorSubcoreMesh(core_axis_name="c", subcore_axis_name="s")
    xs = pl.kernel(
        _sc_scatter,
        mesh=sc_mesh,
        out_type=jax.ShapeDtypeStruct((NTOT, D), jnp.float32),
        scratch_types=[
            pltpu.VMEM((4, 32), jnp.int32),
            pltpu.VMEM((32, D), jnp.float32),
            pltpu.SemaphoreType.DMA,
        ],
    )(x, dest_sc)

    h = pl.pallas_call(
        _gu_body,
        grid_spec=pltpu.PrefetchScalarGridSpec(
            num_scalar_prefetch=2,
            grid=(NT,),
            in_specs=[
                pl.BlockSpec((M, D), lambda j, te, na: (j, 0)),
                pl.BlockSpec((1, D, DFF), lambda j, te, na: (te[j, 0], 0, 0)),
                pl.BlockSpec((1, D, DFF), lambda j, te, na: (te[j, 0], 0, 0)),
            ],
            out_specs=pl.BlockSpec((M, DFF), lambda j, te, na: (j, 0)),
        ),
        out_shape=jax.ShapeDtypeStruct((NTOT, DFF), jnp.float32),
        compiler_params=pltpu.CompilerParams(
            dimension_semantics=("arbitrary",)),
    )(te, na, xs, Wg, Wu)

    ys = pl.pallas_call(
        _dn_body,
        grid_spec=pltpu.PrefetchScalarGridSpec(
            num_scalar_prefetch=2,
            grid=(NT,),
            in_specs=[
                pl.BlockSpec((M, DFF), lambda j, te, na: (j, 0)),
                pl.BlockSpec((1, DFF, D), lambda j, te, na: (te[j, 0], 0, 0)),
            ],
            out_specs=pl.BlockSpec((M, D), lambda j, te, na: (j, 0)),
        ),
        out_shape=jax.ShapeDtypeStruct((NTOT, D), jnp.float32),
        compiler_params=pltpu.CompilerParams(
            dimension_semantics=("arbitrary",)),
    )(te, na, h, Wd)

    g0, g1 = pl.kernel(
        _sc_gather,
        mesh=sc_mesh,
        out_type=(
            jax.ShapeDtypeStruct((T, D), jnp.float32),
            jax.ShapeDtypeStruct((T, D), jnp.float32),
        ),
        scratch_types=[
            pltpu.VMEM((32,), jnp.int32),
            pltpu.VMEM((32, D), jnp.float32),
            pltpu.SemaphoreType.DMA,
        ],
    )(ys, pos_sc)

    final = pl.pallas_call(
        _shared_body,
        grid=(T // BTS, NF),
        in_specs=[
            pl.BlockSpec((BTS, D), lambda i, f: (i, 0)),
            pl.BlockSpec((D, BF), lambda i, f: (0, f)),
            pl.BlockSpec((D, BF), lambda i, f: (0, f)),
            pl.BlockSpec((BF, D), lambda i, f: (f, 0)),
            pl.BlockSpec((BTS, 1), lambda i, f: (i, 0)),
            pl.BlockSpec((BTS, D), lambda i, f: (i, 0)),
            pl.BlockSpec((BTS, D), lambda i, f: (i, 0)),
            pl.BlockSpec((BTS, TOP_K), lambda i, f: (i, 0)),
        ],
        out_specs=pl.BlockSpec((BTS, D), lambda i, f: (i, 0)),
        out_shape=jax.ShapeDtypeStruct((T, D), jnp.float32),
        compiler_params=pltpu.CompilerParams(
            dimension_semantics=("parallel", "arbitrary")),
    )(x, sWg, sWu, sWd, gate, g0, g1, w)

    return final.reshape(B, S, D), logits


# speedup vs baseline: 1.1735x; 1.1735x over previous
import jax
import jax.numpy as jnp
from jax import lax
from jax.experimental import pallas as pl
from jax.experimental.pallas import tpu as pltpu
from jax.experimental.pallas import tpu_sc as plsc

B, S, D = 1, 2048, 2048
E, TOP_K = 8, 2
DFF = 1408
DSH = 5632
T = B * S
A = T * TOP_K

M = 256
NT = A // M + E
NTOT = NT * M

BTS = 512
BF = 512
NF = DSH // BF

NW = 32


def _router_body(x_ref, gw_ref, sgw_ref, logits_ref, w_ref, gate_ref,
                 dest_ref, te_ref, na_ref):
    xf = x_ref[...]
    x = xf.astype(jnp.bfloat16)
    logits = lax.dot_general(
        x, gw_ref[...].astype(jnp.bfloat16), (((1,), (0,)), ((), ())),
        preferred_element_type=jnp.float32)
    logits_ref[...] = logits
    gate_ref[...] = jax.nn.sigmoid(
        jnp.sum(xf * sgw_ref[...], axis=-1, keepdims=True))
    mx = jnp.max(logits, axis=-1, keepdims=True)
    ex = jnp.exp(logits - mx)
    p = ex / jnp.sum(ex, axis=-1, keepdims=True)
    lane = lax.broadcasted_iota(jnp.int32, p.shape, 1)
    m1 = jnp.max(p, axis=-1, keepdims=True)
    i1 = jnp.min(jnp.where(p >= m1, lane, E), axis=-1, keepdims=True)
    oh1 = lane == i1
    p2 = jnp.where(oh1, -1.0, p)
    m2 = jnp.max(p2, axis=-1, keepdims=True)
    i2 = jnp.min(jnp.where(p2 >= m2, lane, E), axis=-1, keepdims=True)
    oh2 = lane == i2
    denom = m1 + m2
    w_ref[...] = jnp.concatenate([m1 / denom, m2 / denom], axis=1)

    oh = jnp.concatenate([oh1, oh2], axis=0).astype(jnp.float32)
    inc = oh
    sh = 1
    while sh < A:
        inc = inc + jnp.concatenate(
            [jnp.zeros((sh, E), jnp.float32), inc[:-sh]], axis=0)
        sh *= 2
    r = inc - oh
    counts = inc[A - 1:A, :]
    tpe = jnp.floor((counts + (M - 1)) / M)
    ts_inc = tpe
    sh = 1
    while sh < E:
        ts_inc = ts_inc + jnp.concatenate(
            [jnp.zeros((1, sh), jnp.float32), ts_inc[:, :-sh]], axis=1)
        sh *= 2
    ts = ts_inc - tpe
    dest = jnp.sum(oh * (ts * M + r), axis=1, keepdims=True)
    dest_ref[...] = dest.astype(jnp.int32)

    tile = lax.broadcasted_iota(jnp.int32, (NT, E), 0).astype(jnp.float32)
    cmp = (tile >= jnp.broadcast_to(ts, (NT, E))).astype(jnp.float32)
    te = jnp.sum(cmp, axis=1, keepdims=True) - 1.0
    te_ref[...] = jnp.maximum(te, 0.0).astype(jnp.int32)
    na_ref[...] = ts_inc[:, E - 1:E].astype(jnp.int32)


def _conv_body(in_ref, out_ref):
    out_ref[...] = in_ref[...].astype(jnp.bfloat16)


def _to_bf16(a, blk):
    n = a.shape[0] // blk
    return pl.pallas_call(
        _conv_body,
        grid=(n,),
        in_specs=[pl.BlockSpec((blk,) + a.shape[1:],
                               lambda i: (i,) + (0,) * (a.ndim - 1))],
        out_specs=pl.BlockSpec((blk,) + a.shape[1:],
                               lambda i: (i,) + (0,) * (a.ndim - 1)),
        out_shape=jax.ShapeDtypeStruct(a.shape, jnp.bfloat16),
        compiler_params=pltpu.CompilerParams(
            dimension_semantics=("arbitrary",)),
    )(a)


def _gu_body(te_ref, na_ref, xs_ref, wg_ref, wu_ref, h_ref):
    j = pl.program_id(0)

    @pl.when(j < na_ref[0, 0])
    def _():
        xb = xs_ref[...]
        g = jnp.dot(xb, wg_ref[0], preferred_element_type=jnp.float32)
        u = jnp.dot(xb, wu_ref[0], preferred_element_type=jnp.float32)
        h_ref[...] = g * jax.nn.sigmoid(g) * u


def _dn_body(te_ref, na_ref, h_ref, wd_ref, ys_ref):
    j = pl.program_id(0)

    @pl.when(j < na_ref[0, 0])
    def _():
        h = h_ref[...]
        ys_ref[...] = jax.lax.dot_general(
            h, wd_ref[0], (((1,), (0,)), ((), ())),
            preferred_element_type=jnp.float32)


def _shared_body(x_ref, wg_ref, wu_ref, wd_ref, gate_ref, g0_ref, g1_ref,
                 w_ref, out_ref):
    f = pl.program_id(1)
    x = x_ref[...].astype(jnp.bfloat16)
    g = jnp.dot(x, wg_ref[...], preferred_element_type=jnp.float32)
    u = jnp.dot(x, wu_ref[...], preferred_element_type=jnp.float32)
    h = (g * jax.nn.sigmoid(g) * u).astype(jnp.bfloat16)
    partial = jnp.dot(h, wd_ref[...], preferred_element_type=jnp.float32)
    prev = jnp.where(f == 0, jnp.zeros_like(partial), out_ref[...])
    acc = prev + partial

    @pl.when(f < NF - 1)
    def _():
        out_ref[...] = acc

    @pl.when(f == NF - 1)
    def _():
        wv = w_ref[...]
        out_ref[...] = (gate_ref[...] * acc + wv[:, 0:1] * g0_ref[...]
                        + wv[:, 1:2] * g1_ref[...])


def _sc_scatter(x_ref, dest_ref, xs_ref, idx_v, rows_v, sem):
    w = lax.axis_index("s") * 2 + lax.axis_index("c")
    t0 = (w % 16) * 128
    pltpu.sync_copy(dest_ref.at[w], idx_v)
    for c in range(4):
        pltpu.sync_copy(x_ref.at[pl.ds(t0 + c * 32, 32)], rows_v)
        pltpu.async_copy(rows_v, xs_ref.at[idx_v.at[c]], sem).wait()


def _sc_gather(ys_ref, pos_ref, g0_ref, g1_ref, idx_v, rows_v, sem):
    w = lax.axis_index("s") * 2 + lax.axis_index("c")
    for k in range(2):
        gout = g0_ref if k == 0 else g1_ref
        for c in range(2):
            j = w * 2 + c
            pltpu.sync_copy(pos_ref.at[k, j], idx_v)
            pltpu.async_copy(ys_ref.at[idx_v], rows_v, sem).wait()
            pltpu.sync_copy(rows_v, gout.at[pl.ds(j * 32, 32)])


@jax.jit
def kernel(hidden_states, gate_w, Wg, Wu, Wd, sWg, sWu, sWd, shared_gate_w):
    x = hidden_states.reshape(T, D)
    sgw_t = shared_gate_w.reshape(1, D)
    logits, w, gate, dest, te, na = pl.pallas_call(
        _router_body,
        out_shape=(
            jax.ShapeDtypeStruct((T, E), jnp.float32),
            jax.ShapeDtypeStruct((T, TOP_K), jnp.float32),
            jax.ShapeDtypeStruct((T, 1), jnp.float32),
            jax.ShapeDtypeStruct((A, 1), jnp.int32),
            jax.ShapeDtypeStruct((NT, 1), jnp.int32),
            jax.ShapeDtypeStruct((1, 1), jnp.int32),
        ),
    )(x, gate_w, sgw_t)

    dest_flat = dest.reshape(A)
    dest_sc = dest_flat.reshape(NW, 4, 32)
    pos_sc = dest_flat.reshape(2, 64, 32)

    sc_mesh = plsc.VectorSubcoreMesh(core_axis_name="c", subcore_axis_name="s")
    xs = pl.kernel(
        _sc_scatter,
        mesh=sc_mesh,
        out_type=jax.ShapeDtypeStruct((NTOT, D), jnp.float32),
        scratch_types=[
            pltpu.VMEM((4, 32), jnp.int32),
            pltpu.VMEM((32, D), jnp.float32),
            pltpu.SemaphoreType.DMA,
        ],
    )(x, dest_sc)

    h = pl.pallas_call(
        _gu_body,
        grid_spec=pltpu.PrefetchScalarGridSpec(
            num_scalar_prefetch=2,
            grid=(NT,),
            in_specs=[
                pl.BlockSpec((M, D), lambda j, te, na: (j, 0)),
                pl.BlockSpec((1, D, DFF), lambda j, te, na: (te[j, 0], 0, 0)),
                pl.BlockSpec((1, D, DFF), lambda j, te, na: (te[j, 0], 0, 0)),
            ],
            out_specs=pl.BlockSpec((M, DFF), lambda j, te, na: (j, 0)),
        ),
        out_shape=jax.ShapeDtypeStruct((NTOT, DFF), jnp.float32),
        compiler_params=pltpu.CompilerParams(
            dimension_semantics=("arbitrary",)),
    )(te, na, xs, Wg, Wu)

    ys = pl.pallas_call(
        _dn_body,
        grid_spec=pltpu.PrefetchScalarGridSpec(
            num_scalar_prefetch=2,
            grid=(NT,),
            in_specs=[
                pl.BlockSpec((M, DFF), lambda j, te, na: (j, 0)),
                pl.BlockSpec((1, DFF, D), lambda j, te, na: (te[j, 0], 0, 0)),
            ],
            out_specs=pl.BlockSpec((M, D), lambda j, te, na: (j, 0)),
        ),
        out_shape=jax.ShapeDtypeStruct((NTOT, D), jnp.float32),
        compiler_params=pltpu.CompilerParams(
            dimension_semantics=("arbitrary",)),
    )(te, na, h, Wd)

    sWg_bf = _to_bf16(sWg, D // 4)
    sWu_bf = _to_bf16(sWu, D // 4)
    sWd_bf = _to_bf16(sWd, DSH // 4)

    g0, g1 = pl.kernel(
        _sc_gather,
        mesh=sc_mesh,
        out_type=(
            jax.ShapeDtypeStruct((T, D), jnp.float32),
            jax.ShapeDtypeStruct((T, D), jnp.float32),
        ),
        scratch_types=[
            pltpu.VMEM((32,), jnp.int32),
            pltpu.VMEM((32, D), jnp.float32),
            pltpu.SemaphoreType.DMA,
        ],
    )(ys, pos_sc)

    final = pl.pallas_call(
        _shared_body,
        grid=(T // BTS, NF),
        in_specs=[
            pl.BlockSpec((BTS, D), lambda i, f: (i, 0)),
            pl.BlockSpec((D, BF), lambda i, f: (0, f)),
            pl.BlockSpec((D, BF), lambda i, f: (0, f)),
            pl.BlockSpec((BF, D), lambda i, f: (f, 0)),
            pl.BlockSpec((BTS, 1), lambda i, f: (i, 0)),
            pl.BlockSpec((BTS, D), lambda i, f: (i, 0)),
            pl.BlockSpec((BTS, D), lambda i, f: (i, 0)),
            pl.BlockSpec((BTS, TOP_K), lambda i, f: (i, 0)),
        ],
        out_specs=pl.BlockSpec((BTS, D), lambda i, f: (i, 0)),
        out_shape=jax.ShapeDtypeStruct((T, D), jnp.float32),
        compiler_params=pltpu.CompilerParams(
            dimension_semantics=("parallel", "arbitrary")),
    )(x, sWg_bf, sWu_bf, sWd_bf, gate, g0, g1, w)

    return final.reshape(B, S, D), logits


# speedup vs baseline: 1.1807x; 1.0061x over previous
import jax
import jax.numpy as jnp
from jax import lax
from jax.experimental import pallas as pl
from jax.experimental.pallas import tpu as pltpu
from jax.experimental.pallas import tpu_sc as plsc

B, S, D = 1, 2048, 2048
E, TOP_K = 8, 2
DFF = 1408
DSH = 5632
T = B * S
A = T * TOP_K

M = 256
NT = A // M + E
NTOT = NT * M

BTS = 512
BF = 512
NF = DSH // BF

NW = 32


def _router_body(x_ref, gw_ref, sgw_ref, logits_ref, w_ref, gate_ref,
                 dest_ref, te_ref, na_ref):
    xf = x_ref[...]
    x = xf.astype(jnp.bfloat16)
    logits = lax.dot_general(
        x, gw_ref[...].astype(jnp.bfloat16), (((1,), (0,)), ((), ())),
        preferred_element_type=jnp.float32)
    logits_ref[...] = logits
    gate_ref[...] = jax.nn.sigmoid(
        jnp.sum(xf * sgw_ref[...], axis=-1, keepdims=True))
    mx = jnp.max(logits, axis=-1, keepdims=True)
    ex = jnp.exp(logits - mx)
    p = ex / jnp.sum(ex, axis=-1, keepdims=True)
    lane = lax.broadcasted_iota(jnp.int32, p.shape, 1)
    m1 = jnp.max(p, axis=-1, keepdims=True)
    i1 = jnp.min(jnp.where(p >= m1, lane, E), axis=-1, keepdims=True)
    oh1 = lane == i1
    p2 = jnp.where(oh1, -1.0, p)
    m2 = jnp.max(p2, axis=-1, keepdims=True)
    i2 = jnp.min(jnp.where(p2 >= m2, lane, E), axis=-1, keepdims=True)
    oh2 = lane == i2
    denom = m1 + m2
    w_ref[...] = jnp.concatenate([m1 / denom, m2 / denom], axis=1)

    oh = jnp.concatenate([oh1, oh2], axis=0).astype(jnp.float32)
    inc = oh
    sh = 1
    while sh < A:
        inc = inc + jnp.concatenate(
            [jnp.zeros((sh, E), jnp.float32), inc[:-sh]], axis=0)
        sh *= 2
    r = inc - oh
    counts = inc[A - 1:A, :]
    tpe = jnp.floor((counts + (M - 1)) / M)
    ts_inc = tpe
    sh = 1
    while sh < E:
        ts_inc = ts_inc + jnp.concatenate(
            [jnp.zeros((1, sh), jnp.float32), ts_inc[:, :-sh]], axis=1)
        sh *= 2
    ts = ts_inc - tpe
    dest = jnp.sum(oh * (ts * M + r), axis=1, keepdims=True)
    dest_ref[...] = dest.astype(jnp.int32)

    tile = lax.broadcasted_iota(jnp.int32, (NT, E), 0).astype(jnp.float32)
    cmp = (tile >= jnp.broadcast_to(ts, (NT, E))).astype(jnp.float32)
    te = jnp.sum(cmp, axis=1, keepdims=True) - 1.0
    te_ref[...] = jnp.maximum(te, 0.0).astype(jnp.int32)
    na_ref[...] = ts_inc[:, E - 1:E].astype(jnp.int32)


def _conv_body(in_ref, out_ref):
    out_ref[...] = in_ref[...].astype(jnp.bfloat16)


def _to_bf16(a, blk):
    n = a.shape[0] // blk
    return pl.pallas_call(
        _conv_body,
        grid=(n,),
        in_specs=[pl.BlockSpec((blk,) + a.shape[1:],
                               lambda i: (i,) + (0,) * (a.ndim - 1))],
        out_specs=pl.BlockSpec((blk,) + a.shape[1:],
                               lambda i: (i,) + (0,) * (a.ndim - 1)),
        out_shape=jax.ShapeDtypeStruct(a.shape, jnp.bfloat16),
        compiler_params=pltpu.CompilerParams(
            dimension_semantics=("arbitrary",)),
    )(a)


def _gu_body(te_ref, na_ref, xs_ref, wg_ref, wu_ref, h_ref):
    j = pl.program_id(0)

    @pl.when(j < na_ref[0, 0])
    def _():
        xb = xs_ref[...]
        g = jnp.dot(xb, wg_ref[0], preferred_element_type=jnp.float32)
        u = jnp.dot(xb, wu_ref[0], preferred_element_type=jnp.float32)
        h_ref[...] = g * jax.nn.sigmoid(g) * u


def _dn_body(te_ref, na_ref, h_ref, wd_ref, ys_ref):
    j = pl.program_id(0)

    @pl.when(j < na_ref[0, 0])
    def _():
        h = h_ref[...]
        ys_ref[...] = jax.lax.dot_general(
            h, wd_ref[0], (((1,), (0,)), ((), ())),
            preferred_element_type=jnp.float32)


def _shared_body(x_ref, wg_ref, wu_ref, wd_ref, gate_ref, g0_ref, g1_ref,
                 w_ref, out_ref):
    f = pl.program_id(1)
    x = x_ref[...].astype(jnp.bfloat16)
    g = jnp.dot(x, wg_ref[...], preferred_element_type=jnp.float32)
    u = jnp.dot(x, wu_ref[...], preferred_element_type=jnp.float32)
    h = (g * jax.nn.sigmoid(g) * u).astype(jnp.bfloat16)
    partial = jnp.dot(h, wd_ref[...], preferred_element_type=jnp.float32)
    prev = jnp.where(f == 0, jnp.zeros_like(partial), out_ref[...])
    acc = prev + partial

    @pl.when(f < NF - 1)
    def _():
        out_ref[...] = acc

    @pl.when(f == NF - 1)
    def _():
        wv = w_ref[...]
        out_ref[...] = (gate_ref[...] * acc + wv[:, 0:1] * g0_ref[...]
                        + wv[:, 1:2] * g1_ref[...])


def _sc_scatter(x_ref, dest_ref, xs_ref, idx_v, rows_v, sem):
    w = lax.axis_index("s") * 2 + lax.axis_index("c")
    t0 = (w % 16) * 128
    pltpu.sync_copy(dest_ref.at[w], idx_v)
    for c in range(4):
        pltpu.sync_copy(x_ref.at[pl.ds(t0 + c * 32, 32)], rows_v)
        pltpu.async_copy(rows_v, xs_ref.at[idx_v.at[c]], sem).wait()


def _sc_gather(ys_ref, pos_ref, g0_ref, g1_ref, idx_v, rows_v, sem):
    w = lax.axis_index("s") * 2 + lax.axis_index("c")
    for k in range(2):
        gout = g0_ref if k == 0 else g1_ref
        for c in range(2):
            j = w * 2 + c
            pltpu.sync_copy(pos_ref.at[k, j], idx_v)
            pltpu.async_copy(ys_ref.at[idx_v], rows_v, sem).wait()
            pltpu.sync_copy(rows_v, gout.at[pl.ds(j * 32, 32)])


@jax.jit
def kernel(hidden_states, gate_w, Wg, Wu, Wd, sWg, sWu, sWd, shared_gate_w):
    x = hidden_states.reshape(T, D)
    sgw_t = shared_gate_w.reshape(1, D)
    logits, w, gate, dest, te, na = pl.pallas_call(
        _router_body,
        out_shape=(
            jax.ShapeDtypeStruct((T, E), jnp.float32),
            jax.ShapeDtypeStruct((T, TOP_K), jnp.float32),
            jax.ShapeDtypeStruct((T, 1), jnp.float32),
            jax.ShapeDtypeStruct((A, 1), jnp.int32),
            jax.ShapeDtypeStruct((NT, 1), jnp.int32),
            jax.ShapeDtypeStruct((1, 1), jnp.int32),
        ),
    )(x, gate_w, sgw_t)

    dest_flat = dest.reshape(A)
    dest_sc = dest_flat.reshape(NW, 4, 32)
    pos_sc = dest_flat.reshape(2, 64, 32)

    sc_mesh = plsc.VectorSubcoreMesh(core_axis_name="c", subcore_axis_name="s")
    xs = pl.kernel(
        _sc_scatter,
        mesh=sc_mesh,
        out_type=jax.ShapeDtypeStruct((NTOT, D), jnp.float32),
        scratch_types=[
            pltpu.VMEM((4, 32), jnp.int32),
            pltpu.VMEM((32, D), jnp.float32),
            pltpu.SemaphoreType.DMA,
        ],
    )(x, dest_sc)

    sWg_bf = _to_bf16(sWg, D // 4)
    xs, sWg_bf = lax.optimization_barrier((xs, sWg_bf))

    h = pl.pallas_call(
        _gu_body,
        grid_spec=pltpu.PrefetchScalarGridSpec(
            num_scalar_prefetch=2,
            grid=(NT,),
            in_specs=[
                pl.BlockSpec((M, D), lambda j, te, na: (j, 0)),
                pl.BlockSpec((1, D, DFF), lambda j, te, na: (te[j, 0], 0, 0)),
                pl.BlockSpec((1, D, DFF), lambda j, te, na: (te[j, 0], 0, 0)),
            ],
            out_specs=pl.BlockSpec((M, DFF), lambda j, te, na: (j, 0)),
        ),
        out_shape=jax.ShapeDtypeStruct((NTOT, DFF), jnp.float32),
        compiler_params=pltpu.CompilerParams(
            dimension_semantics=("arbitrary",)),
    )(te, na, xs, Wg, Wu)

    ys = pl.pallas_call(
        _dn_body,
        grid_spec=pltpu.PrefetchScalarGridSpec(
            num_scalar_prefetch=2,
            grid=(NT,),
            in_specs=[
                pl.BlockSpec((M, DFF), lambda j, te, na: (j, 0)),
                pl.BlockSpec((1, DFF, D), lambda j, te, na: (te[j, 0], 0, 0)),
            ],
            out_specs=pl.BlockSpec((M, D), lambda j, te, na: (j, 0)),
        ),
        out_shape=jax.ShapeDtypeStruct((NTOT, D), jnp.float32),
        compiler_params=pltpu.CompilerParams(
            dimension_semantics=("arbitrary",)),
    )(te, na, h, Wd)

    g0, g1 = pl.kernel(
        _sc_gather,
        mesh=sc_mesh,
        out_type=(
            jax.ShapeDtypeStruct((T, D), jnp.float32),
            jax.ShapeDtypeStruct((T, D), jnp.float32),
        ),
        scratch_types=[
            pltpu.VMEM((32,), jnp.int32),
            pltpu.VMEM((32, D), jnp.float32),
            pltpu.SemaphoreType.DMA,
        ],
    )(ys, pos_sc)

    sWu_bf = _to_bf16(sWu, D // 4)
    sWd_bf = _to_bf16(sWd, DSH // 4)

    final = pl.pallas_call(
        _shared_body,
        grid=(T // BTS, NF),
        in_specs=[
            pl.BlockSpec((BTS, D), lambda i, f: (i, 0)),
            pl.BlockSpec((D, BF), lambda i, f: (0, f)),
            pl.BlockSpec((D, BF), lambda i, f: (0, f)),
            pl.BlockSpec((BF, D), lambda i, f: (f, 0)),
            pl.BlockSpec((BTS, 1), lambda i, f: (i, 0)),
            pl.BlockSpec((BTS, D), lambda i, f: (i, 0)),
            pl.BlockSpec((BTS, D), lambda i, f: (i, 0)),
            pl.BlockSpec((BTS, TOP_K), lambda i, f: (i, 0)),
        ],
        out_specs=pl.BlockSpec((BTS, D), lambda i, f: (i, 0)),
        out_shape=jax.ShapeDtypeStruct((T, D), jnp.float32),
        compiler_params=pltpu.CompilerParams(
            dimension_semantics=("parallel", "arbitrary")),
    )(x, sWg_bf, sWu_bf, sWd_bf, gate, g0, g1, w)

    return final.reshape(B, S, D), logits
